# Initial kernel scaffold; baseline (speedup 1.0000x reference)
import functools

import jax
import jax.numpy as jnp
from jax import lax
from jax.experimental import pallas as pl
from jax.experimental.pallas import tpu as pltpu
from jax.experimental.pallas import tpu_sc as plsc

N_NODES = 10000
N_EDGES = 320000
D_FEAT = 128

NUM_CORES = 2
NUM_SUBCORES = 16
NUM_WORKERS = NUM_CORES * NUM_SUBCORES
EDGES_PER_WORKER = N_EDGES // NUM_WORKERS
LANES = 16


def _histogram_body(tgt_hbm, out_hbm, idx_v, cnt_v):
    c = lax.axis_index("c")
    s = lax.axis_index("s")
    wid = s * NUM_CORES + c
    base = wid * EDGES_PER_WORKER

    pltpu.sync_copy(tgt_hbm.at[pl.ds(base, EDGES_PER_WORKER)], idx_v)

    zeros = jnp.zeros((LANES,), jnp.float32)

    def zero_body(i, _):
        cnt_v[pl.ds(i * LANES, LANES)] = zeros
        return ()

    lax.fori_loop(0, N_NODES // LANES, zero_body, (), unroll=8)

    ones = jnp.ones((LANES,), jnp.float32)

    def hist_body(i, _):
        idx = idx_v[pl.ds(i * LANES, LANES)]
        plsc.addupdate_scatter(cnt_v, [idx], ones)
        return ()

    lax.fori_loop(0, EDGES_PER_WORKER // LANES, hist_body, (), unroll=4)

    pltpu.sync_copy(cnt_v, out_hbm.at[wid])


_histogram = pl.kernel(
    _histogram_body,
    out_type=jax.ShapeDtypeStruct((NUM_WORKERS, N_NODES), jnp.float32),
    mesh=plsc.VectorSubcoreMesh(core_axis_name="c", subcore_axis_name="s"),
    scratch_types=[
        pltpu.VMEM((EDGES_PER_WORKER,), jnp.int32),
        pltpu.VMEM((N_NODES,), jnp.float32),
    ],
    name="edge_target_histogram",
)


ROW_BLOCK = 2000


def _scale_matmul_body(x_ref, w_ref, cnt_ref, o_ref):
    deg = jnp.sum(cnt_ref[...], axis=0)
    y = jnp.dot(x_ref[...], w_ref[...], preferred_element_type=jnp.float32)
    o_ref[...] = y * deg[:, None]


def kernel(edge_index, x, W):
    target = edge_index[1]
    partial_counts = _histogram(target)

    out = pl.pallas_call(
        _scale_matmul_body,
        grid=(N_NODES // ROW_BLOCK,),
        in_specs=[
            pl.BlockSpec((ROW_BLOCK, D_FEAT), lambda i: (i, 0)),
            pl.BlockSpec((D_FEAT, D_FEAT), lambda i: (0, 0)),
            pl.BlockSpec((NUM_WORKERS, ROW_BLOCK), lambda i: (0, i)),
        ],
        out_specs=pl.BlockSpec((ROW_BLOCK, D_FEAT), lambda i: (i, 0)),
        out_shape=jax.ShapeDtypeStruct((N_NODES, D_FEAT), jnp.float32),
    )(x, W, partial_counts)
    return out


# speedup vs baseline: 38.5050x; 38.5050x over previous
import functools

import jax
import jax.numpy as jnp
from jax import lax
from jax.experimental import pallas as pl
from jax.experimental.pallas import tpu as pltpu
from jax.experimental.pallas import tpu_sc as plsc

N_NODES = 10000
N_EDGES = 320000
D_FEAT = 128

NUM_CORES = 2
NUM_SUBCORES = 16
NUM_WORKERS = NUM_CORES * NUM_SUBCORES
EDGES_PER_WORKER = N_EDGES // NUM_WORKERS
LANES = 16

ROW_BLOCK = 2048
N_PAD = 10240


def _histogram_body(tgt_hbm, out_hbm, idx_v, cnt_v):
    c = lax.axis_index("c")
    s = lax.axis_index("s")
    wid = s * NUM_CORES + c
    base = wid * EDGES_PER_WORKER

    pltpu.sync_copy(tgt_hbm.at[pl.ds(base, EDGES_PER_WORKER)], idx_v)

    zeros = jnp.zeros((LANES,), jnp.float32)

    def zero_body(i, _):
        cnt_v[pl.ds(i * LANES, LANES)] = zeros
        return ()

    lax.fori_loop(0, N_PAD // LANES, zero_body, (), unroll=8)

    ones = jnp.ones((LANES,), jnp.float32)

    def hist_body(i, _):
        idx = idx_v[pl.ds(i * LANES, LANES)]
        plsc.addupdate_scatter(cnt_v, [idx], ones)
        return ()

    lax.fori_loop(0, EDGES_PER_WORKER // LANES, hist_body, (), unroll=4)

    pltpu.sync_copy(cnt_v, out_hbm.at[wid])


@functools.cache
def _histogram():
    return pl.kernel(
        _histogram_body,
        out_type=jax.ShapeDtypeStruct((NUM_WORKERS, N_PAD), jnp.float32),
        mesh=plsc.VectorSubcoreMesh(core_axis_name="c", subcore_axis_name="s"),
        scratch_types=[
            pltpu.VMEM((EDGES_PER_WORKER,), jnp.int32),
            pltpu.VMEM((N_PAD,), jnp.float32),
        ],
        compiler_params=pltpu.CompilerParams(needs_layout_passes=False),
        name="edge_target_histogram",
    )


def _scale_matmul_body(x_ref, w_ref, cnt_ref, o_ref):
    deg = jnp.sum(cnt_ref[...], axis=0)
    y = jnp.dot(x_ref[...], w_ref[...], preferred_element_type=jnp.float32)
    o_ref[...] = y * deg[:, None]


def kernel(edge_index, x, W):
    target = edge_index[1]
    partial_counts = _histogram()(target)

    x_pad = jnp.pad(x, ((0, N_PAD - N_NODES), (0, 0)))
    out = pl.pallas_call(
        _scale_matmul_body,
        grid=(N_PAD // ROW_BLOCK,),
        in_specs=[
            pl.BlockSpec((ROW_BLOCK, D_FEAT), lambda i: (i, 0)),
            pl.BlockSpec((D_FEAT, D_FEAT), lambda i: (0, 0)),
            pl.BlockSpec((NUM_WORKERS, ROW_BLOCK), lambda i: (0, i)),
        ],
        out_specs=pl.BlockSpec((ROW_BLOCK, D_FEAT), lambda i: (i, 0)),
        out_shape=jax.ShapeDtypeStruct((N_PAD, D_FEAT), jnp.float32),
    )(x_pad, W, partial_counts)
    return out[:N_NODES]
